# Initial kernel scaffold; baseline (speedup 1.0000x reference)
import jax
import jax.numpy as jnp
from jax.experimental import pallas as pl


def kernel(subword_embeddings, begin_indexes, end_indexes, W1, b1, Wout, bout):
    raise NotImplementedError("write your pallas kernel here")


# speedup vs baseline: 43.6533x; 43.6533x over previous
import functools

import jax
import jax.numpy as jnp
from jax import lax
from jax.experimental import pallas as pl
from jax.experimental.pallas import tpu as pltpu
from jax.experimental.pallas import tpu_sc as plsc

_NC = 2
_NS = 16
_NW = _NC * _NS

_BI = 128
_BJ = 256


def _sc_gather(table, idx):
    (b,) = idx.shape
    d = table.shape[1]
    bpw = b // _NW
    mesh = plsc.VectorSubcoreMesh(core_axis_name="c", subcore_axis_name="s")

    @functools.partial(
        pl.kernel,
        mesh=mesh,
        out_type=jax.ShapeDtypeStruct((b, d), table.dtype),
        scratch_types=[
            pltpu.VMEM((bpw,), jnp.int32),
            pltpu.VMEM((bpw, d), table.dtype),
            pltpu.SemaphoreType.DMA,
        ],
    )
    def gather_kernel(table_hbm, idx_hbm, out_hbm, idx_v, rows_v, sem):
        wid = lax.axis_index("s") * _NC + lax.axis_index("c")
        base = wid * bpw
        pltpu.sync_copy(idx_hbm.at[pl.ds(base, bpw)], idx_v)
        pltpu.async_copy(table_hbm.at[idx_v], rows_v, sem).wait()
        pltpu.sync_copy(rows_v, out_hbm.at[pl.ds(base, bpw)])

    return gather_kernel(table, idx)


def _pair_ffnn_body(gi_ref, gj_ref, wl_ref, wr_ref, b1_ref, wout_ref, bout_ref,
                    out_ref):
    h_dim = wout_ref.shape[0]
    out_dim = wout_ref.shape[1]
    l = jnp.dot(gi_ref[...], wl_ref[...], preferred_element_type=jnp.float32)
    l = l + b1_ref[...]
    r = jnp.dot(gj_ref[...], wr_ref[...], preferred_element_type=jnp.float32)
    h = jnp.maximum(l[:, None, :] + r[None, :, :], 0.0)
    o = jnp.dot(h.reshape(_BI * _BJ, h_dim), wout_ref[...],
                preferred_element_type=jnp.float32)
    o = o + bout_ref[...]
    out_ref[...] = o.reshape(_BI, _BJ, out_dim)


def kernel(subword_embeddings, begin_indexes, end_indexes, W1, b1, Wout, bout):
    m = begin_indexes.shape[0]
    d = subword_embeddings.shape[1]
    h_dim = W1.shape[1]
    out_dim = Wout.shape[1]

    idx = jnp.concatenate([begin_indexes, end_indexes]).astype(jnp.int32)
    rows = _sc_gather(subword_embeddings, idx)
    g = jnp.concatenate([rows[:m], rows[m:]], axis=1)

    wl = W1[: 2 * d]
    wr = W1[2 * d:]
    b1_2d = b1.reshape(1, h_dim)
    bout_2d = bout.reshape(1, out_dim)

    grid = (m // _BI, m // _BJ)
    out = pl.pallas_call(
        _pair_ffnn_body,
        grid=grid,
        in_specs=[
            pl.BlockSpec((_BI, 2 * d), lambda i, j: (i, 0)),
            pl.BlockSpec((_BJ, 2 * d), lambda i, j: (j, 0)),
            pl.BlockSpec((2 * d, h_dim), lambda i, j: (0, 0)),
            pl.BlockSpec((2 * d, h_dim), lambda i, j: (0, 0)),
            pl.BlockSpec((1, h_dim), lambda i, j: (0, 0)),
            pl.BlockSpec((h_dim, out_dim), lambda i, j: (0, 0)),
            pl.BlockSpec((1, out_dim), lambda i, j: (0, 0)),
        ],
        out_specs=pl.BlockSpec((_BI, _BJ, out_dim), lambda i, j: (i, j, 0)),
        out_shape=jax.ShapeDtypeStruct((m, m, out_dim), jnp.float32),
    )(g, g, wl, wr, b1_2d, Wout, bout_2d)
    return out


# speedup vs baseline: 61.7969x; 1.4156x over previous
import functools

import jax
import jax.numpy as jnp
from jax import lax
from jax.experimental import pallas as pl
from jax.experimental.pallas import tpu as pltpu
from jax.experimental.pallas import tpu_sc as plsc

_NC = 2
_NS = 16
_NW = _NC * _NS

_BI = 128
_BJ = 256
_PK = 4


def _sc_gather(table, idx):
    (b,) = idx.shape
    d = table.shape[1]
    bpw = b // _NW
    mesh = plsc.VectorSubcoreMesh(core_axis_name="c", subcore_axis_name="s")

    @functools.partial(
        pl.kernel,
        mesh=mesh,
        out_type=jax.ShapeDtypeStruct((b, d), table.dtype),
        scratch_types=[
            pltpu.VMEM((bpw,), jnp.int32),
            pltpu.VMEM((bpw, d), table.dtype),
            pltpu.SemaphoreType.DMA,
        ],
    )
    def gather_kernel(table_hbm, idx_hbm, out_hbm, idx_v, rows_v, sem):
        wid = lax.axis_index("s") * _NC + lax.axis_index("c")
        base = wid * bpw
        pltpu.sync_copy(idx_hbm.at[pl.ds(base, bpw)], idx_v)
        pltpu.async_copy(table_hbm.at[idx_v], rows_v, sem).wait()
        pltpu.sync_copy(rows_v, out_hbm.at[pl.ds(base, bpw)])

    return gather_kernel(table, idx)


def _pair_ffnn_body(gi_ref, gjp_ref, wlw_ref, wrbd_ref, b1w_ref, woutbd_ref,
                    boutw_ref, out_ref):
    hw = wlw_ref.shape[1]
    ow = woutbd_ref.shape[1]
    bjp = gjp_ref.shape[0]
    l = jnp.dot(gi_ref[...], wlw_ref[...], preferred_element_type=jnp.float32)
    l = l + b1w_ref[...]
    r = jnp.dot(gjp_ref[...], wrbd_ref[...], preferred_element_type=jnp.float32)
    h = jnp.maximum(l[:, None, :] + r[None, :, :], 0.0)
    o = jnp.dot(h.reshape(_BI * bjp, hw), woutbd_ref[...],
                preferred_element_type=jnp.float32)
    o = o + boutw_ref[...]
    out_ref[...] = o.reshape(_BI, bjp, ow)


def kernel(subword_embeddings, begin_indexes, end_indexes, W1, b1, Wout, bout):
    m = begin_indexes.shape[0]
    d = subword_embeddings.shape[1]
    h_dim = W1.shape[1]
    out_dim = Wout.shape[1]
    pk = _PK

    idx = jnp.concatenate([begin_indexes, end_indexes]).astype(jnp.int32)
    rows = _sc_gather(subword_embeddings, idx)
    g = jnp.concatenate([rows[:m], rows[m:]], axis=1)
    gjp = g.reshape(m // pk, pk * 2 * d)

    wl = W1[: 2 * d]
    wr = W1[2 * d:]
    wl_wide = jnp.concatenate([wl] * pk, axis=1)
    wr_bd = jax.scipy.linalg.block_diag(*([wr] * pk))
    wout_bd = jax.scipy.linalg.block_diag(*([Wout] * pk))
    b1_wide = jnp.concatenate([b1] * pk).reshape(1, pk * h_dim)
    bout_wide = jnp.concatenate([bout] * pk).reshape(1, pk * out_dim)

    grid = (m // _BI, m // _BJ)
    out = pl.pallas_call(
        _pair_ffnn_body,
        grid=grid,
        in_specs=[
            pl.BlockSpec((_BI, 2 * d), lambda i, j: (i, 0)),
            pl.BlockSpec((_BJ // pk, pk * 2 * d), lambda i, j: (j, 0)),
            pl.BlockSpec((2 * d, pk * h_dim), lambda i, j: (0, 0)),
            pl.BlockSpec((pk * 2 * d, pk * h_dim), lambda i, j: (0, 0)),
            pl.BlockSpec((1, pk * h_dim), lambda i, j: (0, 0)),
            pl.BlockSpec((pk * h_dim, pk * out_dim), lambda i, j: (0, 0)),
            pl.BlockSpec((1, pk * out_dim), lambda i, j: (0, 0)),
        ],
        out_specs=pl.BlockSpec((_BI, _BJ // pk, pk * out_dim),
                               lambda i, j: (i, j, 0)),
        out_shape=jax.ShapeDtypeStruct((m, m // pk, pk * out_dim), jnp.float32),
    )(g, gjp, wl_wide, wr_bd, b1_wide, wout_bd, bout_wide)
    return out.reshape(m, m, out_dim)
